# Initial kernel scaffold; baseline (speedup 1.0000x reference)
import jax
import jax.numpy as jnp
from jax.experimental import pallas as pl


def kernel(text, emb_table, W, b):
    raise NotImplementedError("write your pallas kernel here")


# speedup vs baseline: 14.2432x; 14.2432x over previous
import functools

import jax
import jax.numpy as jnp
from jax import lax
from jax.experimental import pallas as pl
from jax.experimental.pallas import tpu as pltpu
from jax.experimental.pallas import tpu_sc as plsc

VOCAB = 1000000
E = 32
Y = 50
B, L = 4096, 200
N = B * L

NC, NS = 2, 16
NW = NC * NS
PER_W = N // NW
CHUNK = 512
N_CHUNKS = PER_W // CHUNK


def _gather_body(idx_hbm, table_hbm, out_hbm, idx_v, rows_v, sem):
    wid = lax.axis_index("s") * NC + lax.axis_index("c")
    base = wid * PER_W

    def body(i, carry):
        off = base + i * CHUNK
        pltpu.sync_copy(idx_hbm.at[pl.ds(off, CHUNK)], idx_v)
        pltpu.async_copy(table_hbm.at[idx_v], rows_v, sem).wait()
        pltpu.sync_copy(rows_v, out_hbm.at[pl.ds(off, CHUNK)])
        return carry

    lax.fori_loop(0, N_CHUNKS, body, 0, unroll=False)


def _sc_gather(text_flat, table):
    mesh = plsc.VectorSubcoreMesh(core_axis_name="c", subcore_axis_name="s")
    fn = pl.kernel(
        _gather_body,
        mesh=mesh,
        out_type=jax.ShapeDtypeStruct((N, E), jnp.float32),
        scratch_types=[
            pltpu.VMEM((CHUNK,), jnp.int32),
            pltpu.VMEM((CHUNK, E), jnp.float32),
            pltpu.SemaphoreType.DMA,
        ],
        compiler_params=pltpu.CompilerParams(use_tc_tiling_on_sc=False),
    )
    return fn(text_flat, table)


BB = 64


def _tc_body(emb_ref, wt_ref, b_ref, out_ref):
    e = emb_ref[...].reshape(BB * L, E)
    logits = jnp.dot(e, wt_ref[...], preferred_element_type=jnp.float32)
    logits = logits + b_ref[...]
    l3 = logits.reshape(BB, L, Y)
    m = jnp.max(l3, axis=1, keepdims=True)
    ex = jnp.exp(l3 - m)
    s = jnp.sum(ex, axis=1, keepdims=True)
    out_ref[...] = ex / s


def _tc_softmax(embeds3, wt, b2):
    return pl.pallas_call(
        _tc_body,
        grid=(B // BB,),
        in_specs=[
            pl.BlockSpec((BB, L, E), lambda i: (i, 0, 0)),
            pl.BlockSpec((E, Y), lambda i: (0, 0)),
            pl.BlockSpec((1, Y), lambda i: (0, 0)),
        ],
        out_specs=pl.BlockSpec((BB, L, Y), lambda i: (i, 0, 0)),
        out_shape=jax.ShapeDtypeStruct((B, L, Y), jnp.float32),
    )(embeds3, wt, b2)


def kernel(text, emb_table, W, b):
    text_flat = text.reshape(N).astype(jnp.int32)
    embeds = _sc_gather(text_flat, emb_table)
    embeds3 = embeds.reshape(B, L, E)
    return _tc_softmax(embeds3, W.T, b.reshape(1, Y))


# speedup vs baseline: 15.6981x; 1.1021x over previous
import functools

import jax
import jax.numpy as jnp
from jax import lax
from jax.experimental import pallas as pl
from jax.experimental.pallas import tpu as pltpu
from jax.experimental.pallas import tpu_sc as plsc

VOCAB = 1000000
E = 32
Y = 50
B, L = 4096, 200
N = B * L
N4 = N // 4

NC, NS = 2, 16
NW = NC * NS
PER_W = N // NW
CHUNK = 512
N_CHUNKS = PER_W // CHUNK


def _gather_body(idx_hbm, table_hbm, out_hbm, idx_v, rows_v, sem):
    wid = lax.axis_index("s") * NC + lax.axis_index("c")
    base = wid * PER_W

    def body(i, carry):
        off = base + i * CHUNK
        pltpu.sync_copy(idx_hbm.at[pl.ds(off, CHUNK)], idx_v)
        pltpu.async_copy(table_hbm.at[idx_v], rows_v, sem).wait()
        pltpu.sync_copy(rows_v, out_hbm.at[pl.ds(off, CHUNK)])
        return carry

    lax.fori_loop(0, N_CHUNKS, body, 0, unroll=False)


def _sc_gather(text_flat, table):
    mesh = plsc.VectorSubcoreMesh(core_axis_name="c", subcore_axis_name="s")
    fn = pl.kernel(
        _gather_body,
        mesh=mesh,
        out_type=jax.ShapeDtypeStruct((N, E), jnp.float32),
        scratch_types=[
            pltpu.VMEM((CHUNK,), jnp.int32),
            pltpu.VMEM((CHUNK, E), jnp.float32),
            pltpu.SemaphoreType.DMA,
        ],
        compiler_params=pltpu.CompilerParams(use_tc_tiling_on_sc=False),
    )
    return fn(text_flat, table)


BB = 64
BB4 = BB * L // 4
G = L // 4


def _roll(x, k):
    return jnp.concatenate([x[..., k:], x[..., :k]], axis=-1)


def _tc_body(e_ref, wbd_ref, b4_ref, out_ref):
    e4 = e_ref[...]
    l4 = jnp.dot(e4, wbd_ref[...], preferred_element_type=jnp.float32)
    l4 = l4 + b4_ref[...]
    l3 = l4.reshape(BB, G, 4 * Y)
    m1 = jnp.max(l3, axis=1, keepdims=True)
    m = jnp.maximum(jnp.maximum(m1, _roll(m1, Y)),
                    jnp.maximum(_roll(m1, 2 * Y), _roll(m1, 3 * Y)))
    ex = jnp.exp(l3 - m)
    s1 = jnp.sum(ex, axis=1, keepdims=True)
    s = s1 + _roll(s1, Y) + _roll(s1, 2 * Y) + _roll(s1, 3 * Y)
    out_ref[...] = ex * (1.0 / s)


def _tc_softmax(embeds4, wbd, b4):
    return pl.pallas_call(
        _tc_body,
        grid=(B // BB,),
        in_specs=[
            pl.BlockSpec((BB4, 128), lambda i: (i, 0)),
            pl.BlockSpec((128, 4 * Y), lambda i: (0, 0)),
            pl.BlockSpec((1, 4 * Y), lambda i: (0, 0)),
        ],
        out_specs=pl.BlockSpec((BB, G, 4 * Y), lambda i: (i, 0, 0)),
        out_shape=jax.ShapeDtypeStruct((B, G, 4 * Y), jnp.float32),
    )(embeds4, wbd, b4)


def kernel(text, emb_table, W, b):
    text_flat = text.reshape(N).astype(jnp.int32)
    embeds = _sc_gather(text_flat, emb_table)
    embeds4 = embeds.reshape(N4, 128)
    wbd = jnp.zeros((128, 4 * Y), jnp.float32)
    for q in range(4):
        wbd = wbd.at[q * E:(q + 1) * E, q * Y:(q + 1) * Y].set(W.T)
    b4 = jnp.tile(b, 4).reshape(1, 4 * Y)
    packed = _tc_softmax(embeds4, wbd, b4)
    return packed.reshape(B, L, Y)
